# baseline (speedup 1.0000x reference)
import jax
import jax.numpy as jnp
from jax import lax
from jax.experimental import pallas as pl
from jax.experimental.pallas import tpu as pltpu
from jax.experimental.pallas import tpu_sc as plsc

N = 100000
D = 32
E = 1600000
NC = 2
NS = 16
L = 16
HALF = N // NC
RPT = 3200
RP = NS * RPT
DUMMY = HALF
B = 128
T = 100096
STEPS = T // B
EP = NS * T
BLK = 400


NB = 4


def _seg_body(with_cnt, src_hbm, dst_hbm, x_hbm, agg_out, cnt_out,
              agg_sh, cnt_sh, rows_v, sidx_v, didx_v, lidx_v, zrows_v,
              ones_v, ss, sd, sg, sa, sc_):
    c = lax.axis_index("c")
    s = lax.axis_index("s")
    lo = c * HALF
    zv = jnp.zeros((L,), jnp.float32)
    ov = jnp.ones((L,), jnp.float32)

    def zfill(i, _):
        zrows_v[i, pl.ds(0, L)] = zv
        zrows_v[i, pl.ds(L, L)] = zv
        return 0
    lax.fori_loop(0, B, zfill, 0)
    for i in range(B // L):
        ones_v[pl.ds(i * L, L)] = zv

    r0 = s * RPT
    def zspm(j, _):
        pltpu.sync_copy(zrows_v, agg_sh.at[pl.ds(r0 + j * B, B)])
        return 0
    lax.fori_loop(0, RPT // B, zspm, 0)
    if with_cnt:
        def zcnt(j, _):
            pltpu.sync_copy(ones_v, cnt_sh.at[pl.ds(r0 + j * B, B)])
            return 0
        lax.fori_loop(0, RPT // B, zcnt, 0)
    for i in range(B // L):
        ones_v[pl.ds(i * L, L)] = ov
    plsc.subcore_barrier()

    tbase = s * T

    def issue_idx(j, b):
        off = tbase + j * B
        pltpu.async_copy(src_hbm.at[pl.ds(off, B)], sidx_v.at[b], ss.at[b])
        pltpu.async_copy(dst_hbm.at[pl.ds(off, B)], didx_v.at[b], sd.at[b])

    def wait_scat(b):
        pltpu.make_async_copy(rows_v.at[b], agg_sh.at[lidx_v.at[b]],
                              sa.at[b]).wait()
        if with_cnt:
            pltpu.make_async_copy(ones_v, cnt_sh.at[lidx_v.at[b]],
                                  sc_.at[b]).wait()

    def issue_scat(b):
        pltpu.async_copy(rows_v.at[b], agg_sh.at[lidx_v.at[b]], sa.at[b],
                         add=True)
        if with_cnt:
            pltpu.async_copy(ones_v, cnt_sh.at[lidx_v.at[b]], sc_.at[b],
                             add=True)

    for b in range(2):
        issue_idx(b, b)

    TOT = STEPS + 6
    def body4(jj, _):
        for b4 in range(NB):
            j = jj * NB + b4
            b = b4
            bm2 = (b4 - 2) % NB

            @pl.when((j >= 4) & (j < STEPS + 4))
            def _w():
                wait_scat(b)

            @pl.when(j < STEPS)
            def _g():
                pltpu.make_async_copy(src_hbm.at[pl.ds(0, B)], sidx_v.at[b],
                                      ss.at[b]).wait()
                pltpu.make_async_copy(dst_hbm.at[pl.ds(0, B)], didx_v.at[b],
                                      sd.at[b]).wait()
                pltpu.async_copy(x_hbm.at[pl.ds(0, B)], rows_v.at[b],
                                 sg.at[b])
                for i in range(B // L):
                    d = didx_v[b, pl.ds(i * L, L)]
                    inr = (d >= lo) & (d < lo + HALF)
                    lidx_v[b, pl.ds(i * L, L)] = jnp.where(inr, d - lo, DUMMY)

            @pl.when((j >= 2) & (j < STEPS + 2))
            def _r():
                pltpu.make_async_copy(x_hbm.at[sidx_v.at[bm2]],
                                      rows_v.at[bm2], sg.at[bm2]).wait()
                issue_scat(bm2)

            @pl.when(j + 2 < STEPS)
            def _p():
                issue_idx(j + 2, (b4 + 2) % NB)
        return 0
    lax.fori_loop(0, TOT // NB, body4, 0)
    plsc.subcore_barrier()

    pltpu.sync_copy(agg_sh.at[pl.ds(r0, RPT)],
                    agg_out.at[pl.ds(c * RP + r0, RPT)])
    if with_cnt:
        pltpu.sync_copy(cnt_sh.at[pl.ds(r0, RPT)],
                        cnt_out.at[pl.ds(c * RP + r0, RPT)])


_SC_MESH = plsc.VectorSubcoreMesh(core_axis_name="c", subcore_axis_name="s")


def _seg_cnt_body(src_hbm, dst_hbm, x_hbm, agg_out, cnt_out, *scr):
    _seg_body(True, src_hbm, dst_hbm, x_hbm, agg_out, cnt_out, *scr)


def _seg_nocnt_body(src_hbm, dst_hbm, x_hbm, agg_out, *scr):
    _seg_body(False, src_hbm, dst_hbm, x_hbm, agg_out, None, *scr)


def _sc_scratch(with_cnt):
    return [
        pltpu.VMEM_SHARED((RP, D), jnp.float32),
        (pltpu.VMEM_SHARED((RP,), jnp.float32) if with_cnt else
         pltpu.VMEM((L,), jnp.float32)),
        pltpu.VMEM((NB, B, D), jnp.float32),
        pltpu.VMEM((NB, B), jnp.int32),
        pltpu.VMEM((NB, B), jnp.int32),
        pltpu.VMEM((NB, B), jnp.int32),
        pltpu.VMEM((B, D), jnp.float32),
        pltpu.VMEM((B,), jnp.float32),
    ] + [pltpu.SemaphoreType.DMA((NB,))] * 5


_seg_cnt = pl.kernel(
    _seg_cnt_body,
    out_type=(jax.ShapeDtypeStruct((NC * RP, D), jnp.float32),
              jax.ShapeDtypeStruct((NC * RP,), jnp.float32)),
    mesh=_SC_MESH,
    scratch_types=_sc_scratch(True),
    compiler_params=pltpu.CompilerParams(use_tc_tiling_on_sc=False),
)

_seg_nocnt = pl.kernel(
    _seg_nocnt_body,
    out_type=jax.ShapeDtypeStruct((NC * RP, D), jnp.float32),
    mesh=_SC_MESH,
    scratch_types=_sc_scratch(False),
    compiler_params=pltpu.CompilerParams(use_tc_tiling_on_sc=False),
)


def _dense1_kern(agg_ref, cnt_ref, x_ref, wlT_ref, b_ref, wrT_ref, o_ref):
    mean = agg_ref[...] / jnp.maximum(cnt_ref[...], 1.0)
    h = (jnp.dot(mean, wlT_ref[...], preferred_element_type=jnp.float32)
         + b_ref[...]
         + jnp.dot(x_ref[...], wrT_ref[...], preferred_element_type=jnp.float32))
    o_ref[...] = jnp.maximum(h, 0.0)


def _dense2_kern(agg_ref, cnt_ref, x_ref, wlT_ref, b_ref, wrT_ref, o_ref):
    mean = agg_ref[...] / jnp.maximum(cnt_ref[...], 1.0)
    h = (jnp.dot(mean, wlT_ref[...], preferred_element_type=jnp.float32)
         + b_ref[...]
         + jnp.dot(x_ref[...], wrT_ref[...], preferred_element_type=jnp.float32))
    nrm = jnp.sqrt(jnp.sum(h * h, axis=1, keepdims=True))
    o_ref[...] = h / jnp.maximum(nrm, 1e-12)


def _dense(kern, agg_pad, cnt_pad, x, W_l, b_l, W_r):
    pad_map = lambda c, i: (c * (RP // BLK) + i, 0)
    row_map = lambda c, i: (c * (HALF // BLK) + i, 0)
    full_map = lambda c, i: (0, 0)
    return pl.pallas_call(
        kern,
        grid=(NC, HALF // BLK),
        in_specs=[
            pl.BlockSpec((BLK, D), pad_map),
            pl.BlockSpec((BLK, 1), pad_map),
            pl.BlockSpec((BLK, D), row_map),
            pl.BlockSpec((D, D), full_map),
            pl.BlockSpec((1, D), full_map),
            pl.BlockSpec((D, D), full_map),
        ],
        out_specs=pl.BlockSpec((BLK, D), row_map),
        out_shape=jax.ShapeDtypeStruct((N, D), jnp.float32),
    )(agg_pad, cnt_pad.reshape(NC * RP, 1), x, W_l.T, b_l.reshape(1, D), W_r.T)


def kernel(edge_index, emb, W_l1, b_l1, W_r1, W_l2, b_l2, W_r2):
    src = edge_index[0].astype(jnp.int32)
    dst = edge_index[1].astype(jnp.int32)
    pad = EP - E
    src_p = jnp.concatenate([src, jnp.zeros((pad,), jnp.int32)])
    dst_p = jnp.concatenate([dst, jnp.full((pad,), -1, jnp.int32)])
    agg1, cnt = _seg_cnt(src_p, dst_p, emb)
    x1 = _dense(_dense1_kern, agg1, cnt, emb, W_l1, b_l1, W_r1)
    agg2 = _seg_nocnt(src_p, dst_p, x1)
    return _dense(_dense2_kern, agg2, cnt, x1, W_l2, b_l2, W_r2)


# speedup vs baseline: 2.4252x; 2.4252x over previous
import jax
import jax.numpy as jnp
from jax import lax
from jax.experimental import pallas as pl
from jax.experimental.pallas import tpu as pltpu
from jax.experimental.pallas import tpu_sc as plsc

N = 100000
D = 32
E = 1600000
NC = 2
NS = 16
L = 16
HALF = N // NC
RPT = 3200
RP = NS * RPT
DUMMY = HALF
B = 128
T = 100096
STEPS = T // B
EP = NS * T
TB = T + B
REG = NC * NS * TB
NB = 4
NI = 8
BLK = 400

_SC_MESH = plsc.VectorSubcoreMesh(core_axis_name="c", subcore_axis_name="s")
_SC_PARAMS = pltpu.CompilerParams(use_tc_tiling_on_sc=False)


def _part_body(src_hbm, dst_hbm, psrc_out, plidx_out, counts_out,
               sv, dv, sbuf, lbuf, cw_v, si, sdm):
    c = lax.axis_index("c")
    s = lax.axis_index("s")
    lo = c * HALF
    w = c * NS + s
    tbase = s * T
    rbase = w * TB

    def issue(j, b):
        o = tbase + j * B
        pltpu.async_copy(src_hbm.at[pl.ds(o, B)], sv.at[b], si.at[b])
        pltpu.async_copy(dst_hbm.at[pl.ds(o, B)], dv.at[b], sdm.at[b])

    for b in range(2):
        issue(b, b)

    def chunk2(j2, carry):
        for b in range(2):
            j_ = j2 * 2 + b
            off, wp = carry
            pltpu.make_async_copy(src_hbm.at[pl.ds(0, B)], sv.at[b],
                                  si.at[b]).wait()
            pltpu.make_async_copy(dst_hbm.at[pl.ds(0, B)], dv.at[b],
                                  sdm.at[b]).wait()
            for i in range(B // L):
                sn = sv[b, pl.ds(i * L, L)]
                dn = dv[b, pl.ds(i * L, L)]
                m = (dn >= lo) & (dn < lo + HALF)
                mi = jnp.where(m, jnp.int32(1), jnp.int32(0))
                incl = plsc.cumsum(mi)
                pos = off + incl - mi
                plsc.store_scatter(sbuf, [pos], sn, mask=m)
                plsc.store_scatter(lbuf, [pos], dn - lo, mask=m)
                off = off + incl[L - 1]

            @pl.when(j_ + 2 < STEPS)
            def _p():
                issue(j_ + 2, b)

            do_flush = off >= B

            @pl.when(do_flush)
            def _f():
                wpo = pl.multiple_of(rbase + wp, B)
                pltpu.sync_copy(sbuf.at[pl.ds(0, B)],
                                psrc_out.at[pl.ds(wpo, B)])
                pltpu.sync_copy(lbuf.at[pl.ds(0, B)],
                                plidx_out.at[pl.ds(wpo, B)])
                for k in range(B // L):
                    sbuf[pl.ds(k * L, L)] = sbuf[pl.ds(B + k * L, L)]
                    lbuf[pl.ds(k * L, L)] = lbuf[pl.ds(B + k * L, L)]

            carry = (jnp.where(do_flush, off - B, off),
                     jnp.where(do_flush, wp + B, wp))
        return carry

    off, wp = lax.fori_loop(0, STEPS // 2, chunk2,
                            (jnp.int32(0), jnp.int32(0)))

    zi = jnp.zeros((L,), jnp.int32)
    dmv = jnp.full((L,), DUMMY, jnp.int32)
    lane = lax.iota(jnp.int32, L)
    for k in range(B // L):
        plsc.store_scatter(sbuf, [off + k * L + lane], zi)
        plsc.store_scatter(lbuf, [off + k * L + lane], dmv)
    wpo = pl.multiple_of(rbase + wp, B)
    pltpu.sync_copy(sbuf.at[pl.ds(0, B)], psrc_out.at[pl.ds(wpo, B)])
    pltpu.sync_copy(lbuf.at[pl.ds(0, B)], plidx_out.at[pl.ds(wpo, B)])
    cw_v[pl.ds(0, L)] = jnp.full((L,), wp + off, jnp.int32)
    pltpu.sync_copy(cw_v, counts_out.at[w])


_part = pl.kernel(
    _part_body,
    out_type=(jax.ShapeDtypeStruct((REG,), jnp.int32),
              jax.ShapeDtypeStruct((REG,), jnp.int32),
              jax.ShapeDtypeStruct((NC * NS, L), jnp.int32)),
    mesh=_SC_MESH,
    scratch_types=[
        pltpu.VMEM((2, B), jnp.int32),
        pltpu.VMEM((2, B), jnp.int32),
        pltpu.VMEM((2 * B + 2 * L,), jnp.int32),
        pltpu.VMEM((2 * B + 2 * L,), jnp.int32),
        pltpu.VMEM((L,), jnp.int32),
        pltpu.SemaphoreType.DMA((2,)),
        pltpu.SemaphoreType.DMA((2,)),
    ],
    compiler_params=pltpu.CompilerParams(use_tc_tiling_on_sc=False,
                                         needs_layout_passes=False),
)


def _seg_body(with_cnt, psrc_hbm, plidx_hbm, cnts_hbm, x_hbm, agg_out,
              cnt_out, agg_sh, cnt_sh, rows_v, sidx_v, lidx_v, zrows_v,
              ones_v, cv, ss, sl, sg, sa, scn):
    c = lax.axis_index("c")
    s = lax.axis_index("s")
    w = c * NS + s
    rbase = w * TB
    zv = jnp.zeros((L,), jnp.float32)
    ov = jnp.ones((L,), jnp.float32)

    def zfill(i, _):
        zrows_v[i, pl.ds(0, L)] = zv
        zrows_v[i, pl.ds(L, L)] = zv
        return 0
    lax.fori_loop(0, B, zfill, 0)
    for i in range(B // L):
        ones_v[pl.ds(i * L, L)] = zv

    r0 = s * RPT
    def zspm(j, _):
        pltpu.sync_copy(zrows_v, agg_sh.at[pl.ds(r0 + j * B, B)])
        return 0
    lax.fori_loop(0, RPT // B, zspm, 0)
    if with_cnt:
        def zcnt(j, _):
            pltpu.sync_copy(ones_v, cnt_sh.at[pl.ds(r0 + j * B, B)])
            return 0
        lax.fori_loop(0, RPT // B, zcnt, 0)
    for i in range(B // L):
        ones_v[pl.ds(i * L, L)] = ov

    pltpu.sync_copy(cnts_hbm.at[w], cv)
    K = cv[pl.ds(0, L)][0]
    nb = (K + B - 1) // B
    plsc.subcore_barrier()

    def issue_idx(j, b8):
        o = rbase + j * B
        pltpu.async_copy(psrc_hbm.at[pl.ds(o, B)], sidx_v.at[b8], ss.at[b8])
        pltpu.async_copy(plidx_hbm.at[pl.ds(o, B)], lidx_v.at[b8], sl.at[b8])

    def wait_scat(b4, b8):
        pltpu.make_async_copy(rows_v.at[b4], agg_sh.at[lidx_v.at[b8]],
                              sa.at[b4]).wait()
        if with_cnt:
            pltpu.make_async_copy(ones_v, cnt_sh.at[lidx_v.at[b8]],
                                  scn.at[b4]).wait()

    def issue_scat(b4, b8):
        pltpu.async_copy(rows_v.at[b4], agg_sh.at[lidx_v.at[b8]], sa.at[b4],
                         add=True)
        if with_cnt:
            pltpu.async_copy(ones_v, cnt_sh.at[lidx_v.at[b8]], scn.at[b4],
                             add=True)

    @pl.when(nb > 0)
    def _i0():
        issue_idx(0, 0)

    @pl.when(nb > 1)
    def _i1():
        issue_idx(1, 1)

    nit = (nb + 6 + NI - 1) // NI
    def body8(jj, _):
        for b8 in range(NI):
            j = jj * NI + b8
            b4 = b8 % NB

            @pl.when((j >= 4) & (j < nb + 4))
            def _w():
                wait_scat(b4, (b8 + 4) % NI)

            @pl.when(j < nb)
            def _g():
                pltpu.make_async_copy(psrc_hbm.at[pl.ds(0, B)],
                                      sidx_v.at[b8], ss.at[b8]).wait()
                pltpu.make_async_copy(plidx_hbm.at[pl.ds(0, B)],
                                      lidx_v.at[b8], sl.at[b8]).wait()
                pltpu.async_copy(x_hbm.at[sidx_v.at[b8]], rows_v.at[b4],
                                 sg.at[b4])

            @pl.when((j >= 2) & (j < nb + 2))
            def _r():
                pltpu.make_async_copy(x_hbm.at[sidx_v.at[(b8 - 2) % NI]],
                                      rows_v.at[(b4 - 2) % NB],
                                      sg.at[(b4 - 2) % NB]).wait()
                issue_scat((b4 - 2) % NB, (b8 - 2) % NI)

            @pl.when(j + 2 < nb)
            def _p():
                issue_idx(j + 2, (b8 + 2) % NI)
        return 0
    lax.fori_loop(0, nit, body8, 0)
    plsc.subcore_barrier()

    pltpu.sync_copy(agg_sh.at[pl.ds(r0, RPT)],
                    agg_out.at[pl.ds(c * RP + r0, RPT)])
    if with_cnt:
        pltpu.sync_copy(cnt_sh.at[pl.ds(r0, RPT)],
                        cnt_out.at[pl.ds(c * RP + r0, RPT)])


def _seg_cnt_body(psrc, plidx, cnts, x, agg_out, cnt_out, *scr):
    _seg_body(True, psrc, plidx, cnts, x, agg_out, cnt_out, *scr)


def _seg_nocnt_body(psrc, plidx, cnts, x, agg_out, *scr):
    _seg_body(False, psrc, plidx, cnts, x, agg_out, None, *scr)


def _sc_scratch(with_cnt):
    return [
        pltpu.VMEM_SHARED((RP, D), jnp.float32),
        (pltpu.VMEM_SHARED((RP,), jnp.float32) if with_cnt else
         pltpu.VMEM((L,), jnp.float32)),
        pltpu.VMEM((NB, B, D), jnp.float32),
        pltpu.VMEM((NI, B), jnp.int32),
        pltpu.VMEM((NI, B), jnp.int32),
        pltpu.VMEM((B, D), jnp.float32),
        pltpu.VMEM((B,), jnp.float32),
        pltpu.VMEM((L,), jnp.int32),
        pltpu.SemaphoreType.DMA((NI,)),
        pltpu.SemaphoreType.DMA((NI,)),
        pltpu.SemaphoreType.DMA((NB,)),
        pltpu.SemaphoreType.DMA((NB,)),
        pltpu.SemaphoreType.DMA((NB,)),
    ]


_seg_cnt = pl.kernel(
    _seg_cnt_body,
    out_type=(jax.ShapeDtypeStruct((NC * RP, D), jnp.float32),
              jax.ShapeDtypeStruct((NC * RP,), jnp.float32)),
    mesh=_SC_MESH,
    scratch_types=_sc_scratch(True),
    compiler_params=_SC_PARAMS,
)

_seg_nocnt = pl.kernel(
    _seg_nocnt_body,
    out_type=jax.ShapeDtypeStruct((NC * RP, D), jnp.float32),
    mesh=_SC_MESH,
    scratch_types=_sc_scratch(False),
    compiler_params=_SC_PARAMS,
)


def _dense1_kern(agg_ref, cnt_ref, x_ref, wlT_ref, b_ref, wrT_ref, o_ref):
    mean = agg_ref[...] / jnp.maximum(cnt_ref[...], 1.0)
    h = (jnp.dot(mean, wlT_ref[...], preferred_element_type=jnp.float32)
         + b_ref[...]
         + jnp.dot(x_ref[...], wrT_ref[...], preferred_element_type=jnp.float32))
    o_ref[...] = jnp.maximum(h, 0.0)


def _dense2_kern(agg_ref, cnt_ref, x_ref, wlT_ref, b_ref, wrT_ref, o_ref):
    mean = agg_ref[...] / jnp.maximum(cnt_ref[...], 1.0)
    h = (jnp.dot(mean, wlT_ref[...], preferred_element_type=jnp.float32)
         + b_ref[...]
         + jnp.dot(x_ref[...], wrT_ref[...], preferred_element_type=jnp.float32))
    nrm = jnp.sqrt(jnp.sum(h * h, axis=1, keepdims=True))
    o_ref[...] = h / jnp.maximum(nrm, 1e-12)


def _dense(kern, agg_pad, cnt_pad, x, W_l, b_l, W_r):
    pad_map = lambda c, i: (c * (RP // BLK) + i, 0)
    row_map = lambda c, i: (c * (HALF // BLK) + i, 0)
    full_map = lambda c, i: (0, 0)
    return pl.pallas_call(
        kern,
        grid=(NC, HALF // BLK),
        in_specs=[
            pl.BlockSpec((BLK, D), pad_map),
            pl.BlockSpec((BLK, 1), pad_map),
            pl.BlockSpec((BLK, D), row_map),
            pl.BlockSpec((D, D), full_map),
            pl.BlockSpec((1, D), full_map),
            pl.BlockSpec((D, D), full_map),
        ],
        out_specs=pl.BlockSpec((BLK, D), row_map),
        out_shape=jax.ShapeDtypeStruct((N, D), jnp.float32),
    )(agg_pad, cnt_pad.reshape(NC * RP, 1), x, W_l.T, b_l.reshape(1, D), W_r.T)


def kernel(edge_index, emb, W_l1, b_l1, W_r1, W_l2, b_l2, W_r2):
    src = edge_index[0].astype(jnp.int32)
    dst = edge_index[1].astype(jnp.int32)
    pad = EP - E
    src_p = jnp.concatenate([src, jnp.zeros((pad,), jnp.int32)])
    dst_p = jnp.concatenate([dst, jnp.full((pad,), -1, jnp.int32)])
    psrc, plidx, cnts = _part(src_p, dst_p)
    agg1, cnt = _seg_cnt(psrc, plidx, cnts, emb)
    x1 = _dense(_dense1_kern, agg1, cnt, emb, W_l1, b_l1, W_r1)
    agg2 = _seg_nocnt(psrc, plidx, cnts, x1)
    return _dense(_dense2_kern, agg2, cnt, x1, W_l2, b_l2, W_r2)


# speedup vs baseline: 3.6627x; 1.5103x over previous
import jax
import jax.numpy as jnp
from jax import lax
from jax.experimental import pallas as pl
from jax.experimental.pallas import tpu as pltpu
from jax.experimental.pallas import tpu_sc as plsc

N = 100000
D = 32
E = 1600000
NC = 2
NS = 16
L = 16
HALF = N // NC
RPT = HALF // NS
RSH = 50048
DUMMY = HALF
ZR = 125
B = 128
TS = E // NS
FULL = TS // B
TB = 100224
REG = NC * NS * TB
NB = 4
NI = 8
NP = N // 4
BLKP = 1000

_SC_MESH = plsc.VectorSubcoreMesh(core_axis_name="c", subcore_axis_name="s")
_SC_PARAMS = pltpu.CompilerParams(use_tc_tiling_on_sc=False)


def _part_body(ei_hbm, psrc_out, plidx_out, counts_out,
               sv, dv, sbuf, lbuf, cw_v, si, sdm):
    c = lax.axis_index("c")
    s = lax.axis_index("s")
    lo = c * HALF
    w = c * NS + s
    tbase = s * TS
    rbase = w * TB
    lane = lax.iota(jnp.int32, L)

    def issue(o, b):
        pltpu.async_copy(ei_hbm.at[0, pl.ds(o, B)], sv.at[b], si.at[b])
        pltpu.async_copy(ei_hbm.at[1, pl.ds(o, B)], dv.at[b], sdm.at[b])

    def wait_idx(b):
        pltpu.make_async_copy(ei_hbm.at[0, pl.ds(0, B)], sv.at[b],
                              si.at[b]).wait()
        pltpu.make_async_copy(ei_hbm.at[1, pl.ds(0, B)], dv.at[b],
                              sdm.at[b]).wait()

    def compact(b, off, extra_mask):
        for i in range(B // L):
            sn = sv[b, pl.ds(i * L, L)]
            dn = dv[b, pl.ds(i * L, L)]
            m = (dn >= lo) & (dn < lo + HALF)
            if extra_mask is not None:
                m = m & extra_mask[i]
            mi = jnp.where(m, jnp.int32(1), jnp.int32(0))
            incl = plsc.cumsum(mi)
            pos = off + incl - mi
            plsc.store_scatter(sbuf, [pos], sn, mask=m)
            plsc.store_scatter(lbuf, [pos], dn - lo, mask=m)
            off = off + incl[L - 1]
        return off

    def flush(off, wp):
        do_flush = off >= B

        @pl.when(do_flush)
        def _f():
            wpo = pl.multiple_of(rbase + wp, B)
            pltpu.sync_copy(sbuf.at[pl.ds(0, B)], psrc_out.at[pl.ds(wpo, B)])
            pltpu.sync_copy(lbuf.at[pl.ds(0, B)],
                            plidx_out.at[pl.ds(wpo, B)])
            for k in range(B // L):
                sbuf[pl.ds(k * L, L)] = sbuf[pl.ds(B + k * L, L)]
                lbuf[pl.ds(k * L, L)] = lbuf[pl.ds(B + k * L, L)]

        return (jnp.where(do_flush, off - B, off),
                jnp.where(do_flush, wp + B, wp))

    for b in range(2):
        issue(tbase + b * B, b)

    def chunk2(j2, carry):
        for b in range(2):
            j_ = j2 * 2 + b
            off, wp = carry
            wait_idx(b)
            off = compact(b, off, None)

            @pl.when(j_ + 2 < FULL)
            def _p():
                issue(tbase + (j_ + 2) * B, b)

            carry = flush(off, wp)
        return carry

    off, wp = lax.fori_loop(0, FULL // 2, chunk2,
                            (jnp.int32(0), jnp.int32(0)))
    wait_idx(0)
    off = compact(0, off, None)
    off, wp = flush(off, wp)
    to = tbase + TS - B
    pltpu.sync_copy(ei_hbm.at[0, pl.ds(to, B)], sv.at[0])
    pltpu.sync_copy(ei_hbm.at[1, pl.ds(to, B)], dv.at[0])
    tail_keep = [(lane + i * L) >= (B - (TS - FULL * B))
                 for i in range(B // L)]
    off = compact(0, off, tail_keep)
    off, wp = flush(off, wp)

    zi = jnp.zeros((L,), jnp.int32)
    dmv = jnp.full((L,), DUMMY, jnp.int32)
    lane = lax.iota(jnp.int32, L)
    for k in range(B // L):
        plsc.store_scatter(sbuf, [off + k * L + lane], zi)
        plsc.store_scatter(lbuf, [off + k * L + lane], dmv)
    wpo = pl.multiple_of(rbase + wp, B)
    pltpu.sync_copy(sbuf.at[pl.ds(0, B)], psrc_out.at[pl.ds(wpo, B)])
    pltpu.sync_copy(lbuf.at[pl.ds(0, B)], plidx_out.at[pl.ds(wpo, B)])
    cw_v[pl.ds(0, L)] = jnp.full((L,), wp + off, jnp.int32)
    pltpu.sync_copy(cw_v, counts_out.at[w])


_part = pl.kernel(
    _part_body,
    out_type=(jax.ShapeDtypeStruct((REG,), jnp.int32),
              jax.ShapeDtypeStruct((REG,), jnp.int32),
              jax.ShapeDtypeStruct((NC * NS, L), jnp.int32)),
    mesh=_SC_MESH,
    scratch_types=[
        pltpu.VMEM((2, B), jnp.int32),
        pltpu.VMEM((2, B), jnp.int32),
        pltpu.VMEM((2 * B + 2 * L,), jnp.int32),
        pltpu.VMEM((2 * B + 2 * L,), jnp.int32),
        pltpu.VMEM((L,), jnp.int32),
        pltpu.SemaphoreType.DMA((2,)),
        pltpu.SemaphoreType.DMA((2,)),
    ],
    compiler_params=pltpu.CompilerParams(use_tc_tiling_on_sc=False,
                                         needs_layout_passes=False),
)


def _seg_body(with_cnt, psrc_hbm, plidx_hbm, cnts_hbm, x_hbm, agg_out,
              cnt_out, agg_sh, cnt_sh, rows_v, sidx_v, lidx_v, zrows_v,
              ones_v, cv, ss, sl, sg, sa, scn):
    c = lax.axis_index("c")
    s = lax.axis_index("s")
    w = c * NS + s
    rbase = w * TB
    zv = jnp.zeros((L,), jnp.float32)
    ov = jnp.ones((L,), jnp.float32)

    def zfill(i, _):
        zrows_v[i, pl.ds(0, L)] = zv
        zrows_v[i, pl.ds(L, L)] = zv
        return 0
    lax.fori_loop(0, ZR, zfill, 0)
    for i in range(B // L):
        ones_v[pl.ds(i * L, L)] = zv

    r0 = s * RPT
    def zspm(j, _):
        pltpu.sync_copy(zrows_v, agg_sh.at[pl.ds(r0 + j * ZR, ZR)])
        return 0
    lax.fori_loop(0, RPT // ZR, zspm, 0)
    if with_cnt:
        def zcnt(j, _):
            q = s * (RSH // B // NS + 1) + j

            @pl.when(q * B < RSH)
            def _z():
                pltpu.sync_copy(ones_v, cnt_sh.at[pl.ds(q * B, B)])
            return 0
        lax.fori_loop(0, RSH // B // NS + 1, zcnt, 0)
    for i in range(B // L):
        ones_v[pl.ds(i * L, L)] = ov

    pltpu.sync_copy(cnts_hbm.at[w], cv)
    K = cv[pl.ds(0, L)][0]
    nb = (K + B - 1) // B
    plsc.subcore_barrier()

    def issue_idx(j, b8):
        o = rbase + j * B
        pltpu.async_copy(psrc_hbm.at[pl.ds(o, B)], sidx_v.at[b8], ss.at[b8])
        pltpu.async_copy(plidx_hbm.at[pl.ds(o, B)], lidx_v.at[b8], sl.at[b8])

    def wait_scat(b4, b8):
        pltpu.make_async_copy(rows_v.at[b4], agg_sh.at[lidx_v.at[b8]],
                              sa.at[b4]).wait()
        if with_cnt:
            pltpu.make_async_copy(ones_v, cnt_sh.at[lidx_v.at[b8]],
                                  scn.at[b4]).wait()

    def issue_scat(b4, b8):
        pltpu.async_copy(rows_v.at[b4], agg_sh.at[lidx_v.at[b8]], sa.at[b4],
                         add=True)
        if with_cnt:
            pltpu.async_copy(ones_v, cnt_sh.at[lidx_v.at[b8]], scn.at[b4],
                             add=True)

    @pl.when(nb > 0)
    def _i0():
        issue_idx(0, 0)

    @pl.when(nb > 1)
    def _i1():
        issue_idx(1, 1)

    nit = (nb + 6 + NI - 1) // NI
    def body8(jj, _):
        for b8 in range(NI):
            j = jj * NI + b8
            b4 = b8 % NB

            @pl.when((j >= 4) & (j < nb + 4))
            def _w():
                wait_scat(b4, (b8 + 4) % NI)

            @pl.when(j < nb)
            def _g():
                pltpu.make_async_copy(psrc_hbm.at[pl.ds(0, B)],
                                      sidx_v.at[b8], ss.at[b8]).wait()
                pltpu.make_async_copy(plidx_hbm.at[pl.ds(0, B)],
                                      lidx_v.at[b8], sl.at[b8]).wait()
                pltpu.async_copy(x_hbm.at[sidx_v.at[b8]], rows_v.at[b4],
                                 sg.at[b4])

            @pl.when((j >= 2) & (j < nb + 2))
            def _r():
                pltpu.make_async_copy(x_hbm.at[sidx_v.at[(b8 - 2) % NI]],
                                      rows_v.at[(b4 - 2) % NB],
                                      sg.at[(b4 - 2) % NB]).wait()
                issue_scat((b4 - 2) % NB, (b8 - 2) % NI)

            @pl.when(j + 2 < nb)
            def _p():
                issue_idx(j + 2, (b8 + 2) % NI)
        return 0
    lax.fori_loop(0, nit, body8, 0)
    plsc.subcore_barrier()

    pltpu.sync_copy(agg_sh.at[pl.ds(r0, RPT)],
                    agg_out.at[pl.ds(c * HALF + r0, RPT)])
    if with_cnt:
        @pl.when(s == 0)
        def _wc():
            pltpu.sync_copy(cnt_sh.at[pl.ds(0, HALF)],
                            cnt_out.at[pl.ds(c * HALF, HALF)])


def _seg_cnt_body(psrc, plidx, cnts, x, agg_out, cnt_out, *scr):
    _seg_body(True, psrc, plidx, cnts, x, agg_out, cnt_out, *scr)


def _seg_nocnt_body(psrc, plidx, cnts, x, agg_out, *scr):
    _seg_body(False, psrc, plidx, cnts, x, agg_out, None, *scr)


def _sc_scratch(with_cnt):
    return [
        pltpu.VMEM_SHARED((RSH, D), jnp.float32),
        (pltpu.VMEM_SHARED((RSH,), jnp.float32) if with_cnt else
         pltpu.VMEM((L,), jnp.float32)),
        pltpu.VMEM((NB, B, D), jnp.float32),
        pltpu.VMEM((NI, B), jnp.int32),
        pltpu.VMEM((NI, B), jnp.int32),
        pltpu.VMEM((ZR, D), jnp.float32),
        pltpu.VMEM((B,), jnp.float32),
        pltpu.VMEM((L,), jnp.int32),
        pltpu.SemaphoreType.DMA((NI,)),
        pltpu.SemaphoreType.DMA((NI,)),
        pltpu.SemaphoreType.DMA((NB,)),
        pltpu.SemaphoreType.DMA((NB,)),
        pltpu.SemaphoreType.DMA((NB,)),
    ]


_seg_cnt = pl.kernel(
    _seg_cnt_body,
    out_type=(jax.ShapeDtypeStruct((N, D), jnp.float32),
              jax.ShapeDtypeStruct((N,), jnp.float32)),
    mesh=_SC_MESH,
    scratch_types=_sc_scratch(True),
    compiler_params=_SC_PARAMS,
)

_seg_nocnt = pl.kernel(
    _seg_nocnt_body,
    out_type=jax.ShapeDtypeStruct((N, D), jnp.float32),
    mesh=_SC_MESH,
    scratch_types=_sc_scratch(False),
    compiler_params=_SC_PARAMS,
)


def _dense1_kern(agg_ref, inv_ref, x_ref, wl_ref, b_ref, wr_ref, o_ref):
    mean = agg_ref[...] * inv_ref[...]
    h = (jnp.dot(mean, wl_ref[...], preferred_element_type=jnp.float32)
         + b_ref[...]
         + jnp.dot(x_ref[...], wr_ref[...], preferred_element_type=jnp.float32))
    o_ref[...] = jnp.maximum(h, 0.0)


def _dense2_kern(agg_ref, inv_ref, x_ref, wl_ref, b_ref, wr_ref, s_ref,
                 o_ref):
    mean = agg_ref[...] * inv_ref[...]
    h = (jnp.dot(mean, wl_ref[...], preferred_element_type=jnp.float32)
         + b_ref[...]
         + jnp.dot(x_ref[...], wr_ref[...], preferred_element_type=jnp.float32))
    s = jnp.dot(h * h, s_ref[...], preferred_element_type=jnp.float32)
    o_ref[...] = h / jnp.maximum(jnp.sqrt(s), 1e-12)


def _blockdiag4(W):
    z = jnp.zeros((4, D, 4, D), jnp.float32)
    for k in range(4):
        z = z.at[k, :, k, :].set(W)
    return z.reshape(4 * D, 4 * D)


def _dense(kern, aggP, invP, xP, W_l, b_l, W_r, norm):
    row_map = lambda i: (i, 0)
    full_map = lambda i: (0, 0)
    consts = [_blockdiag4(W_l.T), jnp.tile(b_l, 4).reshape(1, 4 * D),
              _blockdiag4(W_r.T)]
    specs = [
        pl.BlockSpec((BLKP, 4 * D), row_map),
        pl.BlockSpec((BLKP, 4 * D), row_map),
        pl.BlockSpec((BLKP, 4 * D), row_map),
        pl.BlockSpec((4 * D, 4 * D), full_map),
        pl.BlockSpec((1, 4 * D), full_map),
        pl.BlockSpec((4 * D, 4 * D), full_map),
    ]
    if norm:
        consts.append(_blockdiag4(jnp.ones((D, D), jnp.float32)))
        specs.append(pl.BlockSpec((4 * D, 4 * D), full_map))
    return pl.pallas_call(
        kern,
        grid=(NP // BLKP,),
        in_specs=specs,
        out_specs=pl.BlockSpec((BLKP, 4 * D), row_map),
        out_shape=jax.ShapeDtypeStruct((NP, 4 * D), jnp.float32),
    )(aggP, invP, xP, *consts)


def kernel(edge_index, emb, W_l1, b_l1, W_r1, W_l2, b_l2, W_r2):
    ei = edge_index.astype(jnp.int32)
    psrc, plidx, cnts = _part(ei)
    agg1, cnt = _seg_cnt(psrc, plidx, cnts, emb)
    invP = jnp.broadcast_to(
        (1.0 / jnp.maximum(cnt, 1.0)).reshape(NP, 4, 1),
        (NP, 4, D)).reshape(NP, 4 * D)
    x1P = _dense(_dense1_kern, agg1.reshape(NP, 4 * D), invP,
                 emb.reshape(NP, 4 * D), W_l1, b_l1, W_r1, norm=False)
    x1 = x1P.reshape(N, D)
    agg2 = _seg_nocnt(psrc, plidx, cnts, x1)
    outP = _dense(_dense2_kern, agg2.reshape(NP, 4 * D), invP, x1P,
                  W_l2, b_l2, W_r2, norm=True)
    return outP.reshape(N, D)


# speedup vs baseline: 3.6758x; 1.0036x over previous
import jax
import jax.numpy as jnp
from jax import lax
from jax.experimental import pallas as pl
from jax.experimental.pallas import tpu as pltpu
from jax.experimental.pallas import tpu_sc as plsc

N = 100000
D = 32
E = 1600000
NC = 2
NS = 16
L = 16
HALF = N // NC
RPT = HALF // NS
RSH = 50048
DUMMY = HALF
ZR = 125
B = 128
TS = E // NS
FULL = TS // B
TB = 100224
REG = NC * NS * TB
NB = 4
NI = 8
NP = N // 4
BLKP = 1000

_SC_MESH = plsc.VectorSubcoreMesh(core_axis_name="c", subcore_axis_name="s")
_SC_PARAMS = pltpu.CompilerParams(use_tc_tiling_on_sc=False)


def _part_body(ei_hbm, psrc_out, plidx_out, counts_out,
               sv, dv, sbuf, lbuf, cw_v, si, sdm):
    c = lax.axis_index("c")
    s = lax.axis_index("s")
    lo = c * HALF
    w = c * NS + s
    tbase = s * TS
    rbase = w * TB
    lane = lax.iota(jnp.int32, L)

    def issue(o, b):
        pltpu.async_copy(ei_hbm.at[0, pl.ds(o, B)], sv.at[b], si.at[b])
        pltpu.async_copy(ei_hbm.at[1, pl.ds(o, B)], dv.at[b], sdm.at[b])

    def wait_idx(b):
        pltpu.make_async_copy(ei_hbm.at[0, pl.ds(0, B)], sv.at[b],
                              si.at[b]).wait()
        pltpu.make_async_copy(ei_hbm.at[1, pl.ds(0, B)], dv.at[b],
                              sdm.at[b]).wait()

    def compact(b, offv, extra_mask):
        for i in range(B // L):
            sn = sv[b, pl.ds(i * L, L)]
            dn = dv[b, pl.ds(i * L, L)]
            m = (dn >= lo) & (dn < lo + HALF)
            if extra_mask is not None:
                m = m & extra_mask[i]
            mi = jnp.where(m, jnp.int32(1), jnp.int32(0))
            incl = plsc.cumsum(mi)
            pos = offv + (incl - mi)
            plsc.store_scatter(sbuf, [pos], sn, mask=m)
            plsc.store_scatter(lbuf, [pos], dn - lo, mask=m)
            offv = offv + plsc.all_reduce_population_count(m)
        return offv

    def flush(offv, wp):
        do_flush = offv[0] >= B

        @pl.when(do_flush)
        def _f():
            wpo = pl.multiple_of(rbase + wp, B)
            pltpu.sync_copy(sbuf.at[pl.ds(0, B)], psrc_out.at[pl.ds(wpo, B)])
            pltpu.sync_copy(lbuf.at[pl.ds(0, B)],
                            plidx_out.at[pl.ds(wpo, B)])
            for k in range(B // L):
                sbuf[pl.ds(k * L, L)] = sbuf[pl.ds(B + k * L, L)]
                lbuf[pl.ds(k * L, L)] = lbuf[pl.ds(B + k * L, L)]

        return (jnp.where(do_flush, offv - B, offv),
                jnp.where(do_flush, wp + B, wp))

    for b in range(2):
        issue(tbase + b * B, b)

    def chunk2(j2, carry):
        for b in range(2):
            j_ = j2 * 2 + b
            off, wp = carry
            wait_idx(b)
            off = compact(b, off, None)

            @pl.when(j_ + 2 < FULL)
            def _p():
                issue(tbase + (j_ + 2) * B, b)

            carry = flush(off, wp)
        return carry

    off, wp = lax.fori_loop(0, FULL // 2, chunk2,
                            (jnp.zeros((L,), jnp.int32), jnp.int32(0)))
    wait_idx(0)
    off = compact(0, off, None)
    off, wp = flush(off, wp)
    to = tbase + TS - B
    pltpu.sync_copy(ei_hbm.at[0, pl.ds(to, B)], sv.at[0])
    pltpu.sync_copy(ei_hbm.at[1, pl.ds(to, B)], dv.at[0])
    tail_keep = [(lane + i * L) >= (B - (TS - FULL * B))
                 for i in range(B // L)]
    off = compact(0, off, tail_keep)
    off, wp = flush(off, wp)

    zi = jnp.zeros((L,), jnp.int32)
    dmv = jnp.full((L,), DUMMY, jnp.int32)
    for k in range(B // L):
        plsc.store_scatter(sbuf, [off + k * L + lane], zi)
        plsc.store_scatter(lbuf, [off + k * L + lane], dmv)
    wpo = pl.multiple_of(rbase + wp, B)
    pltpu.sync_copy(sbuf.at[pl.ds(0, B)], psrc_out.at[pl.ds(wpo, B)])
    pltpu.sync_copy(lbuf.at[pl.ds(0, B)], plidx_out.at[pl.ds(wpo, B)])
    cw_v[pl.ds(0, L)] = wp + off
    pltpu.sync_copy(cw_v, counts_out.at[w])


_part = pl.kernel(
    _part_body,
    out_type=(jax.ShapeDtypeStruct((REG,), jnp.int32),
              jax.ShapeDtypeStruct((REG,), jnp.int32),
              jax.ShapeDtypeStruct((NC * NS, L), jnp.int32)),
    mesh=_SC_MESH,
    scratch_types=[
        pltpu.VMEM((2, B), jnp.int32),
        pltpu.VMEM((2, B), jnp.int32),
        pltpu.VMEM((2 * B + 2 * L,), jnp.int32),
        pltpu.VMEM((2 * B + 2 * L,), jnp.int32),
        pltpu.VMEM((L,), jnp.int32),
        pltpu.SemaphoreType.DMA((2,)),
        pltpu.SemaphoreType.DMA((2,)),
    ],
    compiler_params=pltpu.CompilerParams(use_tc_tiling_on_sc=False,
                                         needs_layout_passes=False),
)


def _seg_body(with_cnt, psrc_hbm, plidx_hbm, cnts_hbm, x_hbm, agg_out,
              cnt_out, agg_sh, cnt_sh, rows_v, sidx_v, lidx_v, zrows_v,
              ones_v, cv, ss, sl, sg, sa, scn):
    c = lax.axis_index("c")
    s = lax.axis_index("s")
    w = c * NS + s
    rbase = w * TB
    zv = jnp.zeros((L,), jnp.float32)
    ov = jnp.ones((L,), jnp.float32)

    def zfill(i, _):
        zrows_v[i, pl.ds(0, L)] = zv
        zrows_v[i, pl.ds(L, L)] = zv
        return 0
    lax.fori_loop(0, ZR, zfill, 0)
    for i in range(B // L):
        ones_v[pl.ds(i * L, L)] = zv

    r0 = s * RPT
    def zspm(j, _):
        pltpu.sync_copy(zrows_v, agg_sh.at[pl.ds(r0 + j * ZR, ZR)])
        return 0
    lax.fori_loop(0, RPT // ZR, zspm, 0)
    if with_cnt:
        def zcnt(j, _):
            q = s * (RSH // B // NS + 1) + j

            @pl.when(q * B < RSH)
            def _z():
                pltpu.sync_copy(ones_v, cnt_sh.at[pl.ds(q * B, B)])
            return 0
        lax.fori_loop(0, RSH // B // NS + 1, zcnt, 0)
    for i in range(B // L):
        ones_v[pl.ds(i * L, L)] = ov

    pltpu.sync_copy(cnts_hbm.at[w], cv)
    K = cv[pl.ds(0, L)][0]
    nb = (K + B - 1) // B
    plsc.subcore_barrier()

    def issue_idx(j, b8):
        o = rbase + j * B
        pltpu.async_copy(psrc_hbm.at[pl.ds(o, B)], sidx_v.at[b8], ss.at[b8])
        pltpu.async_copy(plidx_hbm.at[pl.ds(o, B)], lidx_v.at[b8], sl.at[b8])

    def wait_scat(b4, b8):
        pltpu.make_async_copy(rows_v.at[b4], agg_sh.at[lidx_v.at[b8]],
                              sa.at[b4]).wait()
        if with_cnt:
            pltpu.make_async_copy(ones_v, cnt_sh.at[lidx_v.at[b8]],
                                  scn.at[b4]).wait()

    def issue_scat(b4, b8):
        pltpu.async_copy(rows_v.at[b4], agg_sh.at[lidx_v.at[b8]], sa.at[b4],
                         add=True)
        if with_cnt:
            pltpu.async_copy(ones_v, cnt_sh.at[lidx_v.at[b8]], scn.at[b4],
                             add=True)

    @pl.when(nb > 0)
    def _i0():
        issue_idx(0, 0)

    @pl.when(nb > 1)
    def _i1():
        issue_idx(1, 1)

    nit = (nb + 6 + NI - 1) // NI
    def body8(jj, _):
        for b8 in range(NI):
            j = jj * NI + b8
            b4 = b8 % NB

            @pl.when((j >= 4) & (j < nb + 4))
            def _w():
                wait_scat(b4, (b8 + 4) % NI)

            @pl.when(j < nb)
            def _g():
                pltpu.make_async_copy(psrc_hbm.at[pl.ds(0, B)],
                                      sidx_v.at[b8], ss.at[b8]).wait()
                pltpu.make_async_copy(plidx_hbm.at[pl.ds(0, B)],
                                      lidx_v.at[b8], sl.at[b8]).wait()
                pltpu.async_copy(x_hbm.at[sidx_v.at[b8]], rows_v.at[b4],
                                 sg.at[b4])

            @pl.when((j >= 2) & (j < nb + 2))
            def _r():
                pltpu.make_async_copy(x_hbm.at[sidx_v.at[(b8 - 2) % NI]],
                                      rows_v.at[(b4 - 2) % NB],
                                      sg.at[(b4 - 2) % NB]).wait()
                issue_scat((b4 - 2) % NB, (b8 - 2) % NI)

            @pl.when(j + 2 < nb)
            def _p():
                issue_idx(j + 2, (b8 + 2) % NI)
        return 0
    lax.fori_loop(0, nit, body8, 0)
    plsc.subcore_barrier()

    pltpu.sync_copy(agg_sh.at[pl.ds(r0, RPT)],
                    agg_out.at[pl.ds(c * HALF + r0, RPT)])
    if with_cnt:
        @pl.when(s == 0)
        def _wc():
            pltpu.sync_copy(cnt_sh.at[pl.ds(0, HALF)],
                            cnt_out.at[pl.ds(c * HALF, HALF)])


def _seg_cnt_body(psrc, plidx, cnts, x, agg_out, cnt_out, *scr):
    _seg_body(True, psrc, plidx, cnts, x, agg_out, cnt_out, *scr)


def _seg_nocnt_body(psrc, plidx, cnts, x, agg_out, *scr):
    _seg_body(False, psrc, plidx, cnts, x, agg_out, None, *scr)


def _sc_scratch(with_cnt):
    return [
        pltpu.VMEM_SHARED((RSH, D), jnp.float32),
        (pltpu.VMEM_SHARED((RSH,), jnp.float32) if with_cnt else
         pltpu.VMEM((L,), jnp.float32)),
        pltpu.VMEM((NB, B, D), jnp.float32),
        pltpu.VMEM((NI, B), jnp.int32),
        pltpu.VMEM((NI, B), jnp.int32),
        pltpu.VMEM((ZR, D), jnp.float32),
        pltpu.VMEM((B,), jnp.float32),
        pltpu.VMEM((L,), jnp.int32),
        pltpu.SemaphoreType.DMA((NI,)),
        pltpu.SemaphoreType.DMA((NI,)),
        pltpu.SemaphoreType.DMA((NB,)),
        pltpu.SemaphoreType.DMA((NB,)),
        pltpu.SemaphoreType.DMA((NB,)),
    ]


_seg_cnt = pl.kernel(
    _seg_cnt_body,
    out_type=(jax.ShapeDtypeStruct((N, D), jnp.float32),
              jax.ShapeDtypeStruct((N,), jnp.float32)),
    mesh=_SC_MESH,
    scratch_types=_sc_scratch(True),
    compiler_params=_SC_PARAMS,
)

_seg_nocnt = pl.kernel(
    _seg_nocnt_body,
    out_type=jax.ShapeDtypeStruct((N, D), jnp.float32),
    mesh=_SC_MESH,
    scratch_types=_sc_scratch(False),
    compiler_params=_SC_PARAMS,
)


def _dense1_kern(agg_ref, inv_ref, x_ref, wl_ref, b_ref, wr_ref, o_ref):
    mean = agg_ref[...] * inv_ref[...]
    h = (jnp.dot(mean, wl_ref[...], preferred_element_type=jnp.float32)
         + b_ref[...]
         + jnp.dot(x_ref[...], wr_ref[...], preferred_element_type=jnp.float32))
    o_ref[...] = jnp.maximum(h, 0.0)


def _dense2_kern(agg_ref, inv_ref, x_ref, wl_ref, b_ref, wr_ref, s_ref,
                 o_ref):
    mean = agg_ref[...] * inv_ref[...]
    h = (jnp.dot(mean, wl_ref[...], preferred_element_type=jnp.float32)
         + b_ref[...]
         + jnp.dot(x_ref[...], wr_ref[...], preferred_element_type=jnp.float32))
    s = jnp.dot(h * h, s_ref[...], preferred_element_type=jnp.float32)
    o_ref[...] = h / jnp.maximum(jnp.sqrt(s), 1e-12)


def _blockdiag4(W):
    z = jnp.zeros((4, D, 4, D), jnp.float32)
    for k in range(4):
        z = z.at[k, :, k, :].set(W)
    return z.reshape(4 * D, 4 * D)


def _dense(kern, aggP, invP, xP, W_l, b_l, W_r, norm):
    row_map = lambda i: (i, 0)
    full_map = lambda i: (0, 0)
    consts = [_blockdiag4(W_l.T), jnp.tile(b_l, 4).reshape(1, 4 * D),
              _blockdiag4(W_r.T)]
    specs = [
        pl.BlockSpec((BLKP, 4 * D), row_map),
        pl.BlockSpec((BLKP, 4 * D), row_map),
        pl.BlockSpec((BLKP, 4 * D), row_map),
        pl.BlockSpec((4 * D, 4 * D), full_map),
        pl.BlockSpec((1, 4 * D), full_map),
        pl.BlockSpec((4 * D, 4 * D), full_map),
    ]
    if norm:
        consts.append(_blockdiag4(jnp.ones((D, D), jnp.float32)))
        specs.append(pl.BlockSpec((4 * D, 4 * D), full_map))
    return pl.pallas_call(
        kern,
        grid=(NP // BLKP,),
        in_specs=specs,
        out_specs=pl.BlockSpec((BLKP, 4 * D), row_map),
        out_shape=jax.ShapeDtypeStruct((NP, 4 * D), jnp.float32),
    )(aggP, invP, xP, *consts)


def kernel(edge_index, emb, W_l1, b_l1, W_r1, W_l2, b_l2, W_r2):
    ei = edge_index.astype(jnp.int32)
    psrc, plidx, cnts = _part(ei)
    agg1, cnt = _seg_cnt(psrc, plidx, cnts, emb)
    invP = jnp.broadcast_to(
        (1.0 / jnp.maximum(cnt, 1.0)).reshape(NP, 4, 1),
        (NP, 4, D)).reshape(NP, 4 * D)
    x1P = _dense(_dense1_kern, agg1.reshape(NP, 4 * D), invP,
                 emb.reshape(NP, 4 * D), W_l1, b_l1, W_r1, norm=False)
    x1 = x1P.reshape(N, D)
    agg2 = _seg_nocnt(psrc, plidx, cnts, x1)
    outP = _dense(_dense2_kern, agg2.reshape(NP, 4 * D), invP, x1P,
                  W_l2, b_l2, W_r2, norm=True)
    return outP.reshape(N, D)
